# baseline (speedup 1.0000x reference)
import functools

import jax
import jax.numpy as jnp
from jax import lax
from jax.experimental import pallas as pl
from jax.experimental.pallas import tpu as pltpu
from jax.experimental.pallas import tpu_sc as plsc

N_ATOMS = 10000
N_BONDS = 160000
D = 16
NW = 32
P = 320
NPAD = NW * P
C = 128
G = C // 16

_mesh = plsc.VectorSubcoreMesh(
    core_axis_name="c", subcore_axis_name="s", num_cores=2, num_subcores=16
)

_GATHER_DNUMS = lax.GatherDimensionNumbers(
    offset_dims=(), collapsed_slice_dims=(0,), start_index_map=(0,)
)


def _dyn_gather(v, idx):
    return lax.gather(
        v, idx[:, None], _GATHER_DNUMS, (1,),
        mode=lax.GatherScatterMode.PROMISE_IN_BOUNDS,
    )


@functools.partial(
    pl.kernel,
    out_type=jax.ShapeDtypeStruct((NPAD * D,), jnp.float32),
    mesh=_mesh,
    compiler_params=pltpu.CompilerParams(
        needs_layout_passes=False, use_tc_tiling_on_sc=False
    ),
    scratch_types=[
        pltpu.VMEM((16,), jnp.int32),
        pltpu.VMEM((C,), jnp.int32),
        pltpu.VMEM((C,), jnp.int32),
        pltpu.VMEM((C, D), jnp.float32),
        pltpu.VMEM((C * D * D,), jnp.float32),
        pltpu.VMEM((P * D,), jnp.float32),
        pltpu.SemaphoreType.DMA,
    ],
)
def _sc_message_sum(atom_hbm, bond_hbm, src_hbm, dst_hbm, bounds_hbm,
                    out_hbm, bounds_v, src_v, dst_v, atoms_v, bond_v,
                    win_v, sem):
    wid = lax.axis_index("c") * 16 + lax.axis_index("s")
    lane = lax.iota(jnp.int32, 16)
    m15 = lane == 15

    pltpu.sync_copy(bounds_hbm.at[pl.ds(wid * 16, 16)], bounds_v)
    bv = bounds_v[...]
    e_lo = jnp.sum(jnp.where(lane == 0, bv, 0))
    e_hi = jnp.sum(jnp.where(lane == 1, bv, 0))
    base_atom = wid * P

    def zero_body(j, _):
        win_v[pl.ds(j * 16, 16)] = jnp.zeros((16,), jnp.float32)
        return 0
    lax.fori_loop(0, P, zero_body, 0)

    e8 = jnp.bitwise_and(e_lo, -8)
    n_chunks = jnp.right_shift(e_hi - e8 + (C - 1), 7)

    def chunk_body(ci, _):
        start = e8 + ci * C
        resp_lo = jnp.maximum(e_lo, start)
        resp_hi = jnp.minimum(e_hi, start + C)
        base = pl.multiple_of(jnp.minimum(start, N_BONDS - C), 8)

        pltpu.sync_copy(src_hbm.at[pl.ds(base, C)], src_v)
        pltpu.sync_copy(dst_hbm.at[pl.ds(base, C)], dst_v)
        pltpu.async_copy(atom_hbm.at[src_v], atoms_v, sem).wait()
        pltpu.sync_copy(bond_hbm.at[pl.ds(base * (D * D), C * D * D)], bond_v)

        def group_body(g, _):
            dst_g = dst_v[pl.ds(g * 16, 16)]
            rel16 = (dst_g - base_atom) * 16
            ev = base + g * 16 + lane
            vmi = jnp.where((ev >= resp_lo) & (ev < resp_hi), 1, 0)
            for k in range(16):
                kidx = jnp.full((16,), k, jnp.int32)
                rowsplat = jnp.full((16,), 0, jnp.int32) + (g * 16 + k)
                a_vec = plsc.load_gather(atoms_v, [rowsplat, lane])
                dsp = _dyn_gather(rel16, kidx)
                ok = (_dyn_gather(vmi, kidx) > 0) & m15
                ebase = (g * 16 + k) * (D * D)
                for i in range(D):
                    r = bond_v[pl.ds(ebase + i * 16, 16)]
                    c = plsc.cumsum(r * a_vec)
                    plsc.addupdate_scatter(win_v, [dsp + i], c, mask=ok)
            return 0
        lax.fori_loop(0, G, group_body, 0)
        return 0

    lax.fori_loop(0, n_chunks, chunk_body, 0)

    pltpu.sync_copy(win_v, out_hbm.at[pl.ds(wid * (P * D), P * D)])


def kernel(atom_matrix, bond_matrix, connectivity):
    src = connectivity[:, 1].astype(jnp.int32)
    dst = connectivity[:, 0].astype(jnp.int32)
    bond_flat = bond_matrix.reshape(-1)
    cuts = jnp.arange(NW + 1, dtype=jnp.int32) * P
    edges = jnp.searchsorted(dst, cuts, side="left").astype(jnp.int32)
    bounds = jnp.zeros((NW, 16), jnp.int32)
    bounds = bounds.at[:, 0].set(edges[:-1]).at[:, 1].set(edges[1:])
    out = _sc_message_sum(atom_matrix, bond_flat, src, dst,
                          bounds.reshape(-1))
    return out.reshape(NPAD, D)[:N_ATOMS]


# speedup vs baseline: 1.6272x; 1.6272x over previous
import functools

import jax
import jax.numpy as jnp
from jax import lax
from jax.experimental import pallas as pl
from jax.experimental.pallas import tpu as pltpu
from jax.experimental.pallas import tpu_sc as plsc

N_ATOMS = 10000
N_BONDS = 160000
D = 16
NW = 32
P = 320
NPAD = NW * P
C = 128
G = C // 16

_mesh = plsc.VectorSubcoreMesh(
    core_axis_name="c", subcore_axis_name="s", num_cores=2, num_subcores=16
)

_GATHER_DNUMS = lax.GatherDimensionNumbers(
    offset_dims=(), collapsed_slice_dims=(0,), start_index_map=(0,)
)


def _dyn_gather(v, idx):
    return lax.gather(
        v, idx[:, None], _GATHER_DNUMS, (1,),
        mode=lax.GatherScatterMode.PROMISE_IN_BOUNDS,
    )


@functools.partial(
    pl.kernel,
    out_type=jax.ShapeDtypeStruct((NPAD * D,), jnp.float32),
    mesh=_mesh,
    compiler_params=pltpu.CompilerParams(
        needs_layout_passes=False, use_tc_tiling_on_sc=False
    ),
    scratch_types=[
        pltpu.VMEM((16,), jnp.int32),
        pltpu.VMEM((C,), jnp.int32),
        pltpu.VMEM((C,), jnp.int32),
        pltpu.VMEM((C, D), jnp.float32),
        pltpu.VMEM((C * D * D,), jnp.float32),
        pltpu.VMEM((P * D,), jnp.float32),
        pltpu.SemaphoreType.DMA,
    ],
)
def _sc_message_sum(atom_hbm, bond_hbm, src_hbm, dst_hbm, bounds_hbm,
                    out_hbm, bounds_v, src_v, dst_v, atoms_v, bond_v,
                    win_v, sem):
    wid = lax.axis_index("c") * 16 + lax.axis_index("s")
    lane = lax.iota(jnp.int32, 16)
    m15 = lane == 15

    pltpu.sync_copy(bounds_hbm.at[pl.ds(wid * 16, 16)], bounds_v)
    bv = bounds_v[...]
    e_lo = jnp.sum(jnp.where(lane == 0, bv, 0))
    e_hi = jnp.sum(jnp.where(lane == 1, bv, 0))
    base_atom = wid * P

    @plsc.parallel_loop(0, P, unroll=4)
    def zero_body(j):
        win_v[pl.ds(j * 16, 16)] = jnp.zeros((16,), jnp.float32)

    e8 = jnp.bitwise_and(e_lo, -8)
    n_chunks = jnp.right_shift(e_hi - e8 + (C - 1), 7)

    def chunk_body(ci, _):
        start = e8 + ci * C
        resp_lo = jnp.maximum(e_lo, start)
        resp_hi = jnp.minimum(e_hi, start + C)
        base = pl.multiple_of(jnp.minimum(start, N_BONDS - C), 8)

        pltpu.sync_copy(src_hbm.at[pl.ds(base, C)], src_v)
        pltpu.sync_copy(dst_hbm.at[pl.ds(base, C)], dst_v)
        pltpu.async_copy(atom_hbm.at[src_v], atoms_v, sem).wait()
        pltpu.sync_copy(bond_hbm.at[pl.ds(base * (D * D), C * D * D)], bond_v)

        zero = jnp.zeros((16,), jnp.int32)

        @plsc.parallel_loop(0, C, unroll=4)
        def edge_body(le):
            ev = base + le
            valid = (ev >= resp_lo) & (ev < resp_hi)
            ok = m15 & valid
            lesplat = zero + le
            dsp = plsc.load_gather(dst_v, [lesplat])
            rel16 = (dsp - base_atom) * 16
            a_vec = plsc.load_gather(atoms_v, [lesplat, lane])
            ebase = le * (D * D)
            for i in range(D):
                r = bond_v[pl.ds(ebase + i * 16, 16)]
                c = plsc.cumsum(r * a_vec)
                plsc.addupdate_scatter(win_v, [rel16 + i], c, mask=ok)
        return 0

    lax.fori_loop(0, n_chunks, chunk_body, 0)

    pltpu.sync_copy(win_v, out_hbm.at[pl.ds(wid * (P * D), P * D)])


def kernel(atom_matrix, bond_matrix, connectivity):
    src = connectivity[:, 1].astype(jnp.int32)
    dst = connectivity[:, 0].astype(jnp.int32)
    bond_flat = bond_matrix.reshape(-1)
    cuts = jnp.arange(NW + 1, dtype=jnp.int32) * P
    edges = jnp.searchsorted(dst, cuts, side="left").astype(jnp.int32)
    bounds = jnp.zeros((NW, 16), jnp.int32)
    bounds = bounds.at[:, 0].set(edges[:-1]).at[:, 1].set(edges[1:])
    out = _sc_message_sum(atom_matrix, bond_flat, src, dst,
                          bounds.reshape(-1))
    return out.reshape(NPAD, D)[:N_ATOMS]
